# baseline (speedup 1.0000x reference)
import functools

import jax
import jax.numpy as jnp
from jax import lax
from jax.experimental import pallas as pl
from jax.experimental.pallas import tpu as pltpu
from jax.experimental.pallas import tpu_sc as plsc

N = 10000
N_PAD = 10240
E = 320000
E_PAD = 327680
CHUNK = 128
SUP = 5
EROWS = E_PAD // CHUNK
NCORE = 2
NSUB = 16
NW = NCORE * NSUB
ROWS_PER_W = EROWS // NW
TSUP = ROWS_PER_W // SUP
ZROWS = N_PAD // NSUB
D_IN = 128
DH = 64
NC = 10
NG = 100
NGP = 128
BLKD = 2048
NBLKD = N_PAD // BLKD
BLKF = 1024
NBLKF = N_PAD // BLKF

def _sc_mesh():
    return plsc.VectorSubcoreMesh(core_axis_name="c", subcore_axis_name="s",
                                  num_cores=NCORE, num_subcores=NSUB)


def _degree_body(ed_hbm, out_hbm, ed_v, ones_v, zc_v, cacc):
    c = lax.axis_index("c")
    s = lax.axis_index("s")
    wid = c * NSUB + s
    zeros16 = jnp.zeros((16,), jnp.float32)
    ones16 = jnp.ones((16,), jnp.float32)
    for q in range(CHUNK // 16):
        ones_v[pl.ds(q * 16, 16)] = ones16
    for q in range(ZROWS // 16):
        zc_v[pl.ds(q * 16, 16)] = zeros16
    pltpu.sync_copy(zc_v, cacc.at[pl.ds(s * ZROWS, ZROWS)])
    plsc.subcore_barrier()

    @pl.loop(0, TSUP)
    def _(t):
        base = wid * ROWS_PER_W + t * SUP
        pltpu.sync_copy(ed_hbm.at[pl.ds(base, SUP)], ed_v)
        for j in range(SUP):
            pltpu.sync_copy(ones_v, cacc.at[ed_v.at[j, 2]], add=True)

    plsc.subcore_barrier()
    pltpu.sync_copy(cacc.at[pl.ds(s * ZROWS, ZROWS)],
                    out_hbm.at[c, pl.ds(s * ZROWS, ZROWS)])


def _degree(ed_p):
    return pl.kernel(
        _degree_body,
        out_type=jax.ShapeDtypeStruct((NCORE, N_PAD), jnp.float32),
        mesh=_sc_mesh(),
        scratch_types=[
            pltpu.VMEM((SUP, 3, CHUNK), jnp.int32),
            pltpu.VMEM((CHUNK,), jnp.float32),
            pltpu.VMEM((ZROWS,), jnp.float32),
            pltpu.VMEM_SHARED((N_PAD,), jnp.float32),
        ],
        compiler_params=pltpu.CompilerParams(use_tc_tiling_on_sc=False),
    )(ed_p)


def _edge_agg_body(t_hbm, ed_hbm, out_hbm,
                   ed_v, idx_v, rows_v, acc,
                   sem_st, sem_g, sem_sc):
    c = lax.axis_index("c")
    s = lax.axis_index("s")
    wbase = (c * NSUB + s) * ROWS_PER_W
    zeros16 = jnp.zeros((16,), jnp.float32)

    @pl.loop(0, ZROWS)
    def _(i):
        for q in range(DH // 16):
            rows_v[0, i, pl.ds(q * 16, 16)] = zeros16

    pltpu.sync_copy(rows_v.at[0], acc.at[pl.ds(s * ZROWS, ZROWS)])
    plsc.subcore_barrier()

    def stage(tt, p):
        b = tt * SUP + wbase
        return pltpu.async_copy(ed_hbm.at[pl.ds(b, SUP)], ed_v.at[p], sem_st)

    def idx_compute(p):
        for j in range(SUP):
            for q in range(CHUNK // 16):
                sl = pl.ds(q * 16, 16)
                idx_v[p, j, sl] = (ed_v[p, j, 0, sl]
                                   + (1 - ed_v[p, j, 1, sl]) * N_PAD)

    def gathers(p):
        return [pltpu.async_copy(t_hbm.at[idx_v.at[p, j]],
                                 rows_v.at[p, pl.ds(j * CHUNK, CHUNK)], sem_g)
                for j in range(SUP)]

    def scatter(p, j):
        return pltpu.async_copy(rows_v.at[p, pl.ds(j * CHUNK, CHUNK)],
                                acc.at[ed_v.at[p, j, 2]], sem_sc, add=True)

    @pl.loop(0, TSUP, step=2)
    def _(t):
        st0 = stage(t, 0)
        st1 = stage(t + 1, 1)
        st0.wait()
        idx_compute(0)
        g0 = gathers(0)
        st1.wait()
        idx_compute(1)
        g1 = gathers(1)
        scs = []
        for j in range(SUP):
            g0[j].wait()
            scs.append(scatter(0, j))
        for j in range(SUP):
            g1[j].wait()
            scs.append(scatter(1, j))
        for h in scs:
            h.wait()

    plsc.subcore_barrier()
    pltpu.sync_copy(acc.at[pl.ds(s * ZROWS, ZROWS)],
                    out_hbm.at[c, pl.ds(s * ZROWS, ZROWS)])


def _edge_agg(tbl, ed_p):
    return pl.kernel(
        _edge_agg_body,
        out_type=jax.ShapeDtypeStruct((NCORE, N_PAD, DH), jnp.float32),
        mesh=_sc_mesh(),
        scratch_types=[
            pltpu.VMEM((2, SUP, 3, CHUNK), jnp.int32),
            pltpu.VMEM((2, SUP, CHUNK), jnp.int32),
            pltpu.VMEM((2, SUP * CHUNK, DH), jnp.float32),
            pltpu.VMEM_SHARED((N_PAD, DH), jnp.float32),
            pltpu.SemaphoreType.DMA,
            pltpu.SemaphoreType.DMA,
            pltpu.SemaphoreType.DMA,
        ],
        compiler_params=pltpu.CompilerParams(use_tc_tiling_on_sc=False),
    )(tbl, ed_p)


def _pack_outputs(y_e, y_o, bn, bo, t_ref, r_ref):
    t_ref[0] = jnp.concatenate([y_e[:, :DH] + bn, y_o[:, :DH] + bn], axis=1)
    t_ref[1] = jnp.concatenate([y_e[:, DH:2 * DH] + bo,
                                y_o[:, DH:2 * DH] + bo], axis=1)
    r_ref[...] = jnp.concatenate([y_e[:, 2 * DH:], y_o[:, 2 * DH:]], axis=1)


def _dense0_body(xe_ref, xo_ref, w_ref, bn_ref, bo_ref, t_ref, r_ref):
    y_e = jnp.dot(xe_ref[...], w_ref[...], preferred_element_type=jnp.float32)
    y_o = jnp.dot(xo_ref[...], w_ref[...], preferred_element_type=jnp.float32)
    _pack_outputs(y_e, y_o, bn_ref[...], bo_ref[...], t_ref, r_ref)


def _dense0(x_e, x_o, wc, bn, bo):
    hb = BLKD // 2
    return pl.pallas_call(
        _dense0_body,
        grid=(NBLKD,),
        in_specs=[
            pl.BlockSpec((hb, D_IN), lambda i: (i, 0)),
            pl.BlockSpec((hb, D_IN), lambda i: (i, 0)),
            pl.BlockSpec((D_IN, 3 * DH), lambda i: (0, 0)),
            pl.BlockSpec((1, DH), lambda i: (0, 0)),
            pl.BlockSpec((1, DH), lambda i: (0, 0)),
        ],
        out_specs=[
            pl.BlockSpec((2, hb, 2 * DH), lambda i: (0, i, 0)),
            pl.BlockSpec((hb, 2 * DH), lambda i: (i, 0)),
        ],
        out_shape=[
            jax.ShapeDtypeStruct((2, N_PAD // 2, 2 * DH), jnp.float32),
            jax.ShapeDtypeStruct((N_PAD // 2, 2 * DH), jnp.float32),
        ],
    )(x_e, x_o, wc, bn, bo)


def _post_agg(psum, inv, r, g, be):
    aggv = psum * inv + r
    mu = jnp.mean(aggv, axis=-1, keepdims=True)
    var = jnp.mean((aggv - mu) ** 2, axis=-1, keepdims=True)
    h = (aggv - mu) / jnp.sqrt(var + 1e-5) * g + be
    return jnp.maximum(h, 0.0)


def _halves(p_ref, ce_ref, co_ref, r_ref, g_ref, be_ref):
    p0, p1 = p_ref[0], p_ref[1]
    inv_e = 1.0 / jnp.maximum(ce_ref[0] + ce_ref[1], 1.0)
    inv_o = 1.0 / jnp.maximum(co_ref[0] + co_ref[1], 1.0)
    g, be = g_ref[...], be_ref[...]
    h_e = _post_agg(p0[:, :DH] + p1[:, :DH], inv_e, r_ref[:, :DH], g, be)
    h_o = _post_agg(p0[:, DH:] + p1[:, DH:], inv_o, r_ref[:, DH:], g, be)
    return h_e, h_o


def _combine_body(p_ref, ce_ref, co_ref, r_ref, g_ref, be_ref, w_ref,
                  bn_ref, bo_ref, t_ref, rn_ref):
    h_e, h_o = _halves(p_ref, ce_ref, co_ref, r_ref, g_ref, be_ref)
    y_e = jnp.dot(h_e, w_ref[...], preferred_element_type=jnp.float32)
    y_o = jnp.dot(h_o, w_ref[...], preferred_element_type=jnp.float32)
    _pack_outputs(y_e, y_o, bn_ref[...], bo_ref[...], t_ref, rn_ref)


def _combine(p2, ce, co, r2, g, be, wc, bn, bo):
    hb = BLKD // 2
    one = lambda i: (0, 0)
    return pl.pallas_call(
        _combine_body,
        grid=(NBLKD,),
        in_specs=[
            pl.BlockSpec((2, hb, 2 * DH), lambda i: (0, i, 0)),
            pl.BlockSpec((2, hb, 1), lambda i: (0, i, 0)),
            pl.BlockSpec((2, hb, 1), lambda i: (0, i, 0)),
            pl.BlockSpec((hb, 2 * DH), lambda i: (i, 0)),
            pl.BlockSpec((1, DH), one),
            pl.BlockSpec((1, DH), one),
            pl.BlockSpec((DH, 3 * DH), one),
            pl.BlockSpec((1, DH), one),
            pl.BlockSpec((1, DH), one),
        ],
        out_specs=[
            pl.BlockSpec((2, hb, 2 * DH), lambda i: (0, i, 0)),
            pl.BlockSpec((hb, 2 * DH), lambda i: (i, 0)),
        ],
        out_shape=[
            jax.ShapeDtypeStruct((2, N_PAD // 2, 2 * DH), jnp.float32),
            jax.ShapeDtypeStruct((N_PAD // 2, 2 * DH), jnp.float32),
        ],
    )(p2, ce, co, r2, g, be, wc, bn, bo)


def _dotT(a, b):
    return lax.dot_general(a, b, (((0,), (0,)), ((), ())),
                           preferred_element_type=jnp.float32)


def _final_body(p_ref, ce_ref, co_ref, r_ref, g_ref, be_ref, se_ref, so_ref,
                be2_ref, bo2_ref,
                wlin_ref, gbn_ref, bebn_ref, wo0_ref, wo1_ref, wo2_ref,
                boc_ref, bias_ref,
                outv_ref, xct_ref, loss_ref,
                out_acc, cs_acc):
    i = pl.program_id(0)
    hf = BLKF // 2
    h_e, h_o = _halves(p_ref, ce_ref, co_ref, r_ref, g_ref, be_ref)
    iota = lax.broadcasted_iota(jnp.int32, (hf, NGP), 1)
    bm_e = (be2_ref[...] == iota).astype(jnp.float32)
    bm_o = (bo2_ref[...] == iota).astype(jnp.float32)
    s_e = se_ref[...]
    s_o = so_ref[...]

    @pl.when(i == 0)
    def _():
        out_acc[...] = jnp.zeros_like(out_acc)
        cs_acc[...] = jnp.zeros_like(cs_acc)

    bm = jnp.concatenate([bm_e, bm_o], axis=0)
    cs_acc[...] += _dotT(bm, jnp.concatenate([s_e, s_o], axis=0))
    op_e = jnp.concatenate([s_e[:, c:c + 1] * h_e for c in range(NC)], axis=1)
    op_o = jnp.concatenate([s_o[:, c:c + 1] * h_o for c in range(NC)], axis=1)
    op = jnp.concatenate([op_e, op_o], axis=0)
    out_acc[...] += _dotT(bm, op)

    @pl.when(i == NBLKF - 1)
    def _():
        cs = cs_acc[...]
        wl = wlin_ref[...]
        acc_outv = jnp.zeros((NGP, 1), jnp.float32) + bias_ref[0, 0]
        l1num = jnp.zeros((NGP, 1), jnp.float32)
        for c in range(NC):
            o = out_acc[:, c * DH:(c + 1) * DH]
            mu = jnp.mean(o, axis=-1, keepdims=True)
            var = jnp.mean((o - mu) ** 2, axis=-1, keepdims=True)
            y = (o - mu) / jnp.sqrt(var + 1e-5) * gbn_ref[...] + bebn_ref[...]
            xc = jnp.sum(y * wl, axis=-1, keepdims=True)
            xc = xc * (cs[:, c:c + 1] > 0).astype(jnp.float32)
            xct_ref[:, c:c + 1] = xc
            acc_outv += xc
            l1num += jnp.abs(xc)
        outv_ref[...] = acc_outv
        denom = jnp.sum((cs > 0).astype(jnp.float32) + 1e-7,
                        axis=-1, keepdims=True)
        rowm = (lax.broadcasted_iota(jnp.int32, (NGP, 1), 0) < NG)
        l1 = jnp.sum(rowm.astype(jnp.float32) * l1num / denom) / NG
        reg = (jnp.sum(jnp.abs(wo0_ref[...])) + jnp.sum(jnp.abs(wo1_ref[...]))
               + jnp.sum(jnp.abs(wo2_ref[...])) + jnp.sum(jnp.abs(boc_ref[...])))
        loss_ref[...] = jnp.reshape(0.01 * (reg + l1), (1, 1))


def _final(p2, ce, co, r2, g, be, s_e, s_o, b_e, b_o, wlin, gbn, bebn,
           wo0, wo1, wo2, boc, bias11):
    hf = BLKF // 2
    zero = lambda i: (0, 0)
    return pl.pallas_call(
        _final_body,
        grid=(NBLKF,),
        in_specs=[
            pl.BlockSpec((2, hf, 2 * DH), lambda i: (0, i, 0)),
            pl.BlockSpec((2, hf, 1), lambda i: (0, i, 0)),
            pl.BlockSpec((2, hf, 1), lambda i: (0, i, 0)),
            pl.BlockSpec((hf, 2 * DH), lambda i: (i, 0)),
            pl.BlockSpec((1, DH), zero),
            pl.BlockSpec((1, DH), zero),
            pl.BlockSpec((hf, NC), lambda i: (i, 0)),
            pl.BlockSpec((hf, NC), lambda i: (i, 0)),
            pl.BlockSpec((hf, 1), lambda i: (i, 0)),
            pl.BlockSpec((hf, 1), lambda i: (i, 0)),
            pl.BlockSpec((1, DH), zero),
            pl.BlockSpec((1, DH), zero),
            pl.BlockSpec((1, DH), zero),
            pl.BlockSpec((DH, D_IN), zero),
            pl.BlockSpec((DH, DH), zero),
            pl.BlockSpec((DH, DH), zero),
            pl.BlockSpec((1, 3 * DH), zero),
            pl.BlockSpec((1, 1), zero),
        ],
        out_specs=[
            pl.BlockSpec((NGP, 1), zero),
            pl.BlockSpec((NGP, NC), zero),
            pl.BlockSpec((1, 1), zero),
        ],
        out_shape=[
            jax.ShapeDtypeStruct((NGP, 1), jnp.float32),
            jax.ShapeDtypeStruct((NGP, NC), jnp.float32),
            jax.ShapeDtypeStruct((1, 1), jnp.float32),
        ],
        scratch_shapes=[
            pltpu.VMEM((NGP, NC * DH), jnp.float32),
            pltpu.VMEM((NGP, NC), jnp.float32),
        ],
        compiler_params=pltpu.CompilerParams(
            fuse_transposed_lhs_in_matmul=True),
    )(p2, ce, co, r2, g, be, s_e, s_o, b_e, b_o, wlin, gbn, bebn, wo0,
      wo1, wo2, boc, bias11)


def kernel(x, edge_index, s, batch, mask, W_n0, b_n0, W_o0, b_o0, W_r0, g0,
           be0, W_n1, b_n1, W_o1, b_o1, W_r1, g1, be1, W_n2, b_n2, W_o2,
           b_o2, W_r2, g2, be2, W_lin, bias, g_bn, be_bn):
    f32 = jnp.float32
    src = edge_index[0].astype(jnp.int32)
    dst = edge_index[1].astype(jnp.int32)
    mi = mask.astype(jnp.int32)

    pe = E_PAD - E
    lanes = jnp.arange(pe, dtype=jnp.int32) % CHUNK
    src_p = jnp.concatenate([src, lanes]).reshape(EROWS, CHUNK)
    dst_p = jnp.concatenate([dst, N + lanes]).reshape(EROWS, CHUNK)
    msk_p = jnp.concatenate([mi, jnp.ones((pe,), jnp.int32)]).reshape(
        EROWS, CHUNK)
    ed_p = jnp.stack([src_p, msk_p, dst_p], axis=1)

    x_p = jnp.pad(x.astype(f32), ((0, N_PAD - N), (0, 0)))
    s_p = jnp.pad(s.astype(f32), ((0, N_PAD - N), (0, 0)))
    b_p = jnp.pad(batch.astype(jnp.int32), (0, N_PAD - N),
                  constant_values=NG)
    x_e, x_o = x_p[0::2], x_p[1::2]
    s_e, s_o = s_p[0::2], s_p[1::2]
    b_e = b_p[0::2].reshape(N_PAD // 2, 1)
    b_o = b_p[1::2].reshape(N_PAD // 2, 1)

    cntp = _degree(ed_p)
    ed_p, cntp = lax.optimization_barrier((ed_p, cntp))
    cpair = cntp.reshape(2, N_PAD // 2, 2)
    ce = cpair[:, :, 0:1]
    co = cpair[:, :, 1:2]

    wc0 = jnp.concatenate([W_n0.T, W_o0.T, W_r0.T], axis=1)
    wc1 = jnp.concatenate([W_n1.T, W_o1.T, W_r1.T], axis=1)
    wc2 = jnp.concatenate([W_n2.T, W_o2.T, W_r2.T], axis=1)
    r1 = lambda v: v.reshape(1, -1)
    half = lambda p: p.reshape(2, N_PAD // 2, 2 * DH)

    t0, r0 = _dense0(x_e, x_o, wc0, r1(b_n0), r1(b_o0))
    p0 = _edge_agg(t0.reshape(2 * N_PAD, DH), ed_p)
    t1, rr1 = _combine(half(p0), ce, co, r0, r1(g0), r1(be0), wc1,
                       r1(b_n1), r1(b_o1))
    p1 = _edge_agg(t1.reshape(2 * N_PAD, DH), ed_p)
    t2, rr2 = _combine(half(p1), ce, co, rr1, r1(g1), r1(be1), wc2,
                       r1(b_n2), r1(b_o2))
    p2 = _edge_agg(t2.reshape(2 * N_PAD, DH), ed_p)

    boc = jnp.concatenate([b_o0, b_o1, b_o2]).reshape(1, 3 * DH)
    outv_p, xct_p, loss = _final(half(p2), ce, co, rr2, r1(g2), r1(be2),
                                 s_e, s_o, b_e, b_o,
                                 W_lin.reshape(1, DH), r1(g_bn), r1(be_bn),
                                 W_o0, W_o1, W_o2, boc, bias.reshape(1, 1))

    outv = outv_p[:NG] + 0.0
    losses = loss.reshape(())
    xct = xct_p[:NG].reshape(NG, NC, 1)
    return outv, losses, s, xct


# speedup vs baseline: 1.0424x; 1.0424x over previous
import functools

import jax
import jax.numpy as jnp
from jax import lax
from jax.experimental import pallas as pl
from jax.experimental.pallas import tpu as pltpu
from jax.experimental.pallas import tpu_sc as plsc

N = 10000
N_PAD = 10240
E = 320000
E_PAD = 327680
CHUNK = 128
SUP = 5
EROWS = E_PAD // CHUNK
NCORE = 2
NSUB = 16
NW = NCORE * NSUB
ROWS_PER_W = EROWS // NW
TSUP = ROWS_PER_W // SUP
ZROWS = N_PAD // NSUB
D_IN = 128
DH = 64
NC = 10
NG = 100
NGP = 128
BLKD = 2048
NBLKD = N_PAD // BLKD
BLKF = 1024
NBLKF = N_PAD // BLKF

def _sc_mesh():
    return plsc.VectorSubcoreMesh(core_axis_name="c", subcore_axis_name="s",
                                  num_cores=NCORE, num_subcores=NSUB)


def _degree_body(ed_hbm, out_hbm, dst_v, ones_v, zc_v, cacc):
    c = lax.axis_index("c")
    s = lax.axis_index("s")
    wid = c * NSUB + s
    zeros16 = jnp.zeros((16,), jnp.float32)
    ones16 = jnp.ones((16,), jnp.float32)
    for q in range(CHUNK // 16):
        ones_v[pl.ds(q * 16, 16)] = ones16
    for q in range(ZROWS // 16):
        zc_v[pl.ds(q * 16, 16)] = zeros16
    pltpu.sync_copy(zc_v, cacc.at[pl.ds(s * ZROWS, ZROWS)])
    plsc.subcore_barrier()

    @pl.loop(0, TSUP)
    def _(t):
        base = wid * ROWS_PER_W + t * SUP
        pltpu.sync_copy(ed_hbm.at[pl.ds(base, SUP)], dst_v)
        for j in range(SUP):
            pltpu.sync_copy(ones_v, cacc.at[dst_v.at[j]], add=True)

    plsc.subcore_barrier()
    pltpu.sync_copy(cacc.at[pl.ds(s * ZROWS, ZROWS)],
                    out_hbm.at[c, pl.ds(s * ZROWS, ZROWS)])


def _degree(dst_p):
    return pl.kernel(
        _degree_body,
        out_type=jax.ShapeDtypeStruct((NCORE, N_PAD), jnp.float32),
        mesh=_sc_mesh(),
        scratch_types=[
            pltpu.VMEM((SUP, CHUNK), jnp.int32),
            pltpu.VMEM((CHUNK,), jnp.float32),
            pltpu.VMEM((ZROWS,), jnp.float32),
            pltpu.VMEM_SHARED((N_PAD,), jnp.float32),
        ],
        compiler_params=pltpu.CompilerParams(use_tc_tiling_on_sc=False),
    )(dst_p)


def _edge_agg_body(t_hbm, src_hbm, msk_hbm, dst_hbm, out_hbm,
                   src_v, msk_v, dst_v, idx_v, rows_v, acc,
                   sem_st, sem_g, sem_sc):
    c = lax.axis_index("c")
    s = lax.axis_index("s")
    wbase = (c * NSUB + s) * ROWS_PER_W
    zeros16 = jnp.zeros((16,), jnp.float32)

    @pl.loop(0, ZROWS)
    def _(i):
        for q in range(DH // 16):
            rows_v[0, i, pl.ds(q * 16, 16)] = zeros16

    pltpu.sync_copy(rows_v.at[0], acc.at[pl.ds(s * ZROWS, ZROWS)])
    plsc.subcore_barrier()

    def stage(tt, p):
        b = tt * SUP + wbase
        return [pltpu.async_copy(src_hbm.at[pl.ds(b, SUP)], src_v.at[p], sem_st),
                pltpu.async_copy(msk_hbm.at[pl.ds(b, SUP)], msk_v.at[p], sem_st),
                pltpu.async_copy(dst_hbm.at[pl.ds(b, SUP)], dst_v.at[p], sem_st)]

    def idx_compute(p):
        for j in range(SUP):
            for q in range(CHUNK // 16):
                sl = pl.ds(q * 16, 16)
                idx_v[p, j, sl] = src_v[p, j, sl] + (1 - msk_v[p, j, sl]) * N_PAD

    def gathers(p):
        return [pltpu.async_copy(t_hbm.at[idx_v.at[p, j]],
                                 rows_v.at[p, pl.ds(j * CHUNK, CHUNK)], sem_g)
                for j in range(SUP)]

    def scatter(p, j):
        return pltpu.async_copy(rows_v.at[p, pl.ds(j * CHUNK, CHUNK)],
                                acc.at[dst_v.at[p, j]], sem_sc, add=True)

    @pl.loop(0, TSUP, step=2)
    def _(t):
        st0 = stage(t, 0)
        st1 = stage(t + 1, 1)
        for h in st0:
            h.wait()
        idx_compute(0)
        g0 = gathers(0)
        for h in st1:
            h.wait()
        idx_compute(1)
        g1 = gathers(1)
        scs = []
        for j in range(SUP):
            g0[j].wait()
            scs.append(scatter(0, j))
        for j in range(SUP):
            g1[j].wait()
            scs.append(scatter(1, j))
        for h in scs:
            h.wait()

    plsc.subcore_barrier()
    pltpu.sync_copy(acc.at[pl.ds(s * ZROWS, ZROWS)],
                    out_hbm.at[c, pl.ds(s * ZROWS, ZROWS)])


def _edge_agg(tbl, src_p, msk_p, dst_p):
    return pl.kernel(
        _edge_agg_body,
        out_type=jax.ShapeDtypeStruct((NCORE, N_PAD, DH), jnp.float32),
        mesh=_sc_mesh(),
        scratch_types=[
            pltpu.VMEM((2, SUP, CHUNK), jnp.int32),
            pltpu.VMEM((2, SUP, CHUNK), jnp.int32),
            pltpu.VMEM((2, SUP, CHUNK), jnp.int32),
            pltpu.VMEM((2, SUP, CHUNK), jnp.int32),
            pltpu.VMEM((2, SUP * CHUNK, DH), jnp.float32),
            pltpu.VMEM_SHARED((N_PAD, DH), jnp.float32),
            pltpu.SemaphoreType.DMA,
            pltpu.SemaphoreType.DMA,
            pltpu.SemaphoreType.DMA,
        ],
        compiler_params=pltpu.CompilerParams(use_tc_tiling_on_sc=False),
    )(tbl, src_p, msk_p, dst_p)


def _pack_outputs(y_e, y_o, bn, bo, t_ref, r_ref):
    t_ref[0] = jnp.concatenate([y_e[:, :DH] + bn, y_o[:, :DH] + bn], axis=1)
    t_ref[1] = jnp.concatenate([y_e[:, DH:2 * DH] + bo,
                                y_o[:, DH:2 * DH] + bo], axis=1)
    r_ref[...] = jnp.concatenate([y_e[:, 2 * DH:], y_o[:, 2 * DH:]], axis=1)


def _dense0_body(xe_ref, xo_ref, w_ref, bn_ref, bo_ref, t_ref, r_ref):
    y_e = jnp.dot(xe_ref[...], w_ref[...], preferred_element_type=jnp.float32)
    y_o = jnp.dot(xo_ref[...], w_ref[...], preferred_element_type=jnp.float32)
    _pack_outputs(y_e, y_o, bn_ref[...], bo_ref[...], t_ref, r_ref)


def _dense0(x_e, x_o, wc, bn, bo):
    hb = BLKD // 2
    return pl.pallas_call(
        _dense0_body,
        grid=(NBLKD,),
        in_specs=[
            pl.BlockSpec((hb, D_IN), lambda i: (i, 0)),
            pl.BlockSpec((hb, D_IN), lambda i: (i, 0)),
            pl.BlockSpec((D_IN, 3 * DH), lambda i: (0, 0)),
            pl.BlockSpec((1, DH), lambda i: (0, 0)),
            pl.BlockSpec((1, DH), lambda i: (0, 0)),
        ],
        out_specs=[
            pl.BlockSpec((2, hb, 2 * DH), lambda i: (0, i, 0)),
            pl.BlockSpec((hb, 2 * DH), lambda i: (i, 0)),
        ],
        out_shape=[
            jax.ShapeDtypeStruct((2, N_PAD // 2, 2 * DH), jnp.float32),
            jax.ShapeDtypeStruct((N_PAD // 2, 2 * DH), jnp.float32),
        ],
    )(x_e, x_o, wc, bn, bo)


def _post_agg(psum, inv, r, g, be):
    aggv = psum * inv + r
    mu = jnp.mean(aggv, axis=-1, keepdims=True)
    var = jnp.mean((aggv - mu) ** 2, axis=-1, keepdims=True)
    h = (aggv - mu) / jnp.sqrt(var + 1e-5) * g + be
    return jnp.maximum(h, 0.0)


def _halves(p_ref, ce_ref, co_ref, r_ref, g_ref, be_ref):
    p0, p1 = p_ref[0], p_ref[1]
    inv_e = 1.0 / jnp.maximum(ce_ref[0] + ce_ref[1], 1.0)
    inv_o = 1.0 / jnp.maximum(co_ref[0] + co_ref[1], 1.0)
    g, be = g_ref[...], be_ref[...]
    h_e = _post_agg(p0[:, :DH] + p1[:, :DH], inv_e, r_ref[:, :DH], g, be)
    h_o = _post_agg(p0[:, DH:] + p1[:, DH:], inv_o, r_ref[:, DH:], g, be)
    return h_e, h_o


def _combine_body(p_ref, ce_ref, co_ref, r_ref, g_ref, be_ref, w_ref,
                  bn_ref, bo_ref, t_ref, rn_ref):
    h_e, h_o = _halves(p_ref, ce_ref, co_ref, r_ref, g_ref, be_ref)
    y_e = jnp.dot(h_e, w_ref[...], preferred_element_type=jnp.float32)
    y_o = jnp.dot(h_o, w_ref[...], preferred_element_type=jnp.float32)
    _pack_outputs(y_e, y_o, bn_ref[...], bo_ref[...], t_ref, rn_ref)


def _combine(p2, ce, co, r2, g, be, wc, bn, bo):
    hb = BLKD // 2
    one = lambda i: (0, 0)
    return pl.pallas_call(
        _combine_body,
        grid=(NBLKD,),
        in_specs=[
            pl.BlockSpec((2, hb, 2 * DH), lambda i: (0, i, 0)),
            pl.BlockSpec((2, hb, 1), lambda i: (0, i, 0)),
            pl.BlockSpec((2, hb, 1), lambda i: (0, i, 0)),
            pl.BlockSpec((hb, 2 * DH), lambda i: (i, 0)),
            pl.BlockSpec((1, DH), one),
            pl.BlockSpec((1, DH), one),
            pl.BlockSpec((DH, 3 * DH), one),
            pl.BlockSpec((1, DH), one),
            pl.BlockSpec((1, DH), one),
        ],
        out_specs=[
            pl.BlockSpec((2, hb, 2 * DH), lambda i: (0, i, 0)),
            pl.BlockSpec((hb, 2 * DH), lambda i: (i, 0)),
        ],
        out_shape=[
            jax.ShapeDtypeStruct((2, N_PAD // 2, 2 * DH), jnp.float32),
            jax.ShapeDtypeStruct((N_PAD // 2, 2 * DH), jnp.float32),
        ],
    )(p2, ce, co, r2, g, be, wc, bn, bo)


def _dotT(a, b):
    return lax.dot_general(a, b, (((0,), (0,)), ((), ())),
                           preferred_element_type=jnp.float32)


def _final_body(p_ref, ce_ref, co_ref, r_ref, g_ref, be_ref, se_ref, so_ref,
                be2_ref, bo2_ref,
                wlin_ref, gbn_ref, bebn_ref, wo0_ref, wo1_ref, wo2_ref,
                boc_ref, bias_ref,
                outv_ref, xct_ref, loss_ref,
                out_acc, cs_acc):
    i = pl.program_id(0)
    hf = BLKF // 2
    h_e, h_o = _halves(p_ref, ce_ref, co_ref, r_ref, g_ref, be_ref)
    iota = lax.broadcasted_iota(jnp.int32, (hf, NGP), 1)
    bm_e = (be2_ref[...] == iota).astype(jnp.float32)
    bm_o = (bo2_ref[...] == iota).astype(jnp.float32)
    s_e = se_ref[...]
    s_o = so_ref[...]

    @pl.when(i == 0)
    def _():
        out_acc[...] = jnp.zeros_like(out_acc)
        cs_acc[...] = jnp.zeros_like(cs_acc)

    cs_acc[...] += _dotT(bm_e, s_e) + _dotT(bm_o, s_o)
    op_e = jnp.concatenate([s_e[:, c:c + 1] * h_e for c in range(NC)], axis=1)
    op_o = jnp.concatenate([s_o[:, c:c + 1] * h_o for c in range(NC)], axis=1)
    out_acc[...] += _dotT(bm_e, op_e) + _dotT(bm_o, op_o)

    @pl.when(i == NBLKF - 1)
    def _():
        cs = cs_acc[...]
        wl = wlin_ref[...]
        acc_outv = jnp.zeros((NGP, 1), jnp.float32) + bias_ref[0, 0]
        l1num = jnp.zeros((NGP, 1), jnp.float32)
        for c in range(NC):
            o = out_acc[:, c * DH:(c + 1) * DH]
            mu = jnp.mean(o, axis=-1, keepdims=True)
            var = jnp.mean((o - mu) ** 2, axis=-1, keepdims=True)
            y = (o - mu) / jnp.sqrt(var + 1e-5) * gbn_ref[...] + bebn_ref[...]
            xc = jnp.sum(y * wl, axis=-1, keepdims=True)
            xc = xc * (cs[:, c:c + 1] > 0).astype(jnp.float32)
            xct_ref[:, c:c + 1] = xc
            acc_outv += xc
            l1num += jnp.abs(xc)
        outv_ref[...] = acc_outv
        denom = jnp.sum((cs > 0).astype(jnp.float32) + 1e-7,
                        axis=-1, keepdims=True)
        rowm = (lax.broadcasted_iota(jnp.int32, (NGP, 1), 0) < NG)
        l1 = jnp.sum(rowm.astype(jnp.float32) * l1num / denom) / NG
        reg = (jnp.sum(jnp.abs(wo0_ref[...])) + jnp.sum(jnp.abs(wo1_ref[...]))
               + jnp.sum(jnp.abs(wo2_ref[...])) + jnp.sum(jnp.abs(boc_ref[...])))
        loss_ref[...] = jnp.reshape(0.01 * (reg + l1), (1, 1))


def _final(p2, ce, co, r2, g, be, s_e, s_o, b_e, b_o, wlin, gbn, bebn,
           wo0, wo1, wo2, boc, bias11):
    hf = BLKF // 2
    zero = lambda i: (0, 0)
    return pl.pallas_call(
        _final_body,
        grid=(NBLKF,),
        in_specs=[
            pl.BlockSpec((2, hf, 2 * DH), lambda i: (0, i, 0)),
            pl.BlockSpec((2, hf, 1), lambda i: (0, i, 0)),
            pl.BlockSpec((2, hf, 1), lambda i: (0, i, 0)),
            pl.BlockSpec((hf, 2 * DH), lambda i: (i, 0)),
            pl.BlockSpec((1, DH), zero),
            pl.BlockSpec((1, DH), zero),
            pl.BlockSpec((hf, NC), lambda i: (i, 0)),
            pl.BlockSpec((hf, NC), lambda i: (i, 0)),
            pl.BlockSpec((hf, 1), lambda i: (i, 0)),
            pl.BlockSpec((hf, 1), lambda i: (i, 0)),
            pl.BlockSpec((1, DH), zero),
            pl.BlockSpec((1, DH), zero),
            pl.BlockSpec((1, DH), zero),
            pl.BlockSpec((DH, D_IN), zero),
            pl.BlockSpec((DH, DH), zero),
            pl.BlockSpec((DH, DH), zero),
            pl.BlockSpec((1, 3 * DH), zero),
            pl.BlockSpec((1, 1), zero),
        ],
        out_specs=[
            pl.BlockSpec((NGP, 1), zero),
            pl.BlockSpec((NGP, NC), zero),
            pl.BlockSpec((1, 1), zero),
        ],
        out_shape=[
            jax.ShapeDtypeStruct((NGP, 1), jnp.float32),
            jax.ShapeDtypeStruct((NGP, NC), jnp.float32),
            jax.ShapeDtypeStruct((1, 1), jnp.float32),
        ],
        scratch_shapes=[
            pltpu.VMEM((NGP, NC * DH), jnp.float32),
            pltpu.VMEM((NGP, NC), jnp.float32),
        ],
        compiler_params=pltpu.CompilerParams(
            fuse_transposed_lhs_in_matmul=True),
    )(p2, ce, co, r2, g, be, s_e, s_o, b_e, b_o, wlin, gbn, bebn, wo0,
      wo1, wo2, boc, bias11)


def kernel(x, edge_index, s, batch, mask, W_n0, b_n0, W_o0, b_o0, W_r0, g0,
           be0, W_n1, b_n1, W_o1, b_o1, W_r1, g1, be1, W_n2, b_n2, W_o2,
           b_o2, W_r2, g2, be2, W_lin, bias, g_bn, be_bn):
    f32 = jnp.float32
    src = edge_index[0].astype(jnp.int32)
    dst = edge_index[1].astype(jnp.int32)
    mi = mask.astype(jnp.int32)

    pe = E_PAD - E
    lanes = jnp.arange(pe, dtype=jnp.int32) % CHUNK
    src_p = jnp.concatenate([src, lanes]).reshape(EROWS, CHUNK)
    dst_p = jnp.concatenate([dst, N + lanes]).reshape(EROWS, CHUNK)
    msk_p = jnp.concatenate([mi, jnp.ones((pe,), jnp.int32)]).reshape(
        EROWS, CHUNK)

    x_p = jnp.pad(x.astype(f32), ((0, N_PAD - N), (0, 0)))
    s_p = jnp.pad(s.astype(f32), ((0, N_PAD - N), (0, 0)))
    b_p = jnp.pad(batch.astype(jnp.int32), (0, N_PAD - N),
                  constant_values=NG)
    x_e, x_o = x_p[0::2], x_p[1::2]
    s_e, s_o = s_p[0::2], s_p[1::2]
    b_e = b_p[0::2].reshape(N_PAD // 2, 1)
    b_o = b_p[1::2].reshape(N_PAD // 2, 1)

    cntp = _degree(dst_p)
    src_p, msk_p, dst_p, cntp = lax.optimization_barrier(
        (src_p, msk_p, dst_p, cntp))
    cpair = cntp.reshape(2, N_PAD // 2, 2)
    ce = cpair[:, :, 0:1]
    co = cpair[:, :, 1:2]

    wc0 = jnp.concatenate([W_n0.T, W_o0.T, W_r0.T], axis=1)
    wc1 = jnp.concatenate([W_n1.T, W_o1.T, W_r1.T], axis=1)
    wc2 = jnp.concatenate([W_n2.T, W_o2.T, W_r2.T], axis=1)
    r1 = lambda v: v.reshape(1, -1)
    half = lambda p: p.reshape(2, N_PAD // 2, 2 * DH)

    t0, r0 = _dense0(x_e, x_o, wc0, r1(b_n0), r1(b_o0))
    p0 = _edge_agg(t0.reshape(2 * N_PAD, DH), src_p, msk_p, dst_p)
    t1, rr1 = _combine(half(p0), ce, co, r0, r1(g0), r1(be0), wc1,
                       r1(b_n1), r1(b_o1))
    p1 = _edge_agg(t1.reshape(2 * N_PAD, DH), src_p, msk_p, dst_p)
    t2, rr2 = _combine(half(p1), ce, co, rr1, r1(g1), r1(be1), wc2,
                       r1(b_n2), r1(b_o2))
    p2 = _edge_agg(t2.reshape(2 * N_PAD, DH), src_p, msk_p, dst_p)

    boc = jnp.concatenate([b_o0, b_o1, b_o2]).reshape(1, 3 * DH)
    outv_p, xct_p, loss = _final(half(p2), ce, co, rr2, r1(g2), r1(be2),
                                 s_e, s_o, b_e, b_o,
                                 W_lin.reshape(1, DH), r1(g_bn), r1(be_bn),
                                 W_o0, W_o1, W_o2, boc, bias.reshape(1, 1))

    outv = outv_p[:NG] + 0.0
    losses = loss.reshape(())
    xct = xct_p[:NG].reshape(NG, NC, 1)
    return outv, losses, s, xct


# speedup vs baseline: 1.0444x; 1.0019x over previous
import functools

import jax
import jax.numpy as jnp
from jax import lax
from jax.experimental import pallas as pl
from jax.experimental.pallas import tpu as pltpu
from jax.experimental.pallas import tpu_sc as plsc

N = 10000
N_PAD = 10240
E = 320000
E_PAD = 327680
CHUNK = 128
SUP = 5
EROWS = E_PAD // CHUNK
NCORE = 2
NSUB = 16
NW = NCORE * NSUB
ROWS_PER_W = EROWS // NW
TSUP = ROWS_PER_W // SUP
ZROWS = N_PAD // NSUB
D_IN = 128
DH = 64
NC = 10
NG = 100
NGP = 128
BLKD = 2048
NBLKD = N_PAD // BLKD
BLKF = 2048
NBLKF = N_PAD // BLKF

def _sc_mesh():
    return plsc.VectorSubcoreMesh(core_axis_name="c", subcore_axis_name="s",
                                  num_cores=NCORE, num_subcores=NSUB)


def _degree_body(ed_hbm, out_hbm, dst_v, ones_v, zc_v, cacc):
    c = lax.axis_index("c")
    s = lax.axis_index("s")
    wid = c * NSUB + s
    zeros16 = jnp.zeros((16,), jnp.float32)
    ones16 = jnp.ones((16,), jnp.float32)
    for q in range(CHUNK // 16):
        ones_v[pl.ds(q * 16, 16)] = ones16
    for q in range(ZROWS // 16):
        zc_v[pl.ds(q * 16, 16)] = zeros16
    pltpu.sync_copy(zc_v, cacc.at[pl.ds(s * ZROWS, ZROWS)])
    plsc.subcore_barrier()

    @pl.loop(0, TSUP)
    def _(t):
        base = wid * ROWS_PER_W + t * SUP
        pltpu.sync_copy(ed_hbm.at[pl.ds(base, SUP)], dst_v)
        for j in range(SUP):
            pltpu.sync_copy(ones_v, cacc.at[dst_v.at[j]], add=True)

    plsc.subcore_barrier()
    pltpu.sync_copy(cacc.at[pl.ds(s * ZROWS, ZROWS)],
                    out_hbm.at[c, pl.ds(s * ZROWS, ZROWS)])


def _degree(dst_p):
    return pl.kernel(
        _degree_body,
        out_type=jax.ShapeDtypeStruct((NCORE, N_PAD), jnp.float32),
        mesh=_sc_mesh(),
        scratch_types=[
            pltpu.VMEM((SUP, CHUNK), jnp.int32),
            pltpu.VMEM((CHUNK,), jnp.float32),
            pltpu.VMEM((ZROWS,), jnp.float32),
            pltpu.VMEM_SHARED((N_PAD,), jnp.float32),
        ],
        compiler_params=pltpu.CompilerParams(use_tc_tiling_on_sc=False),
    )(dst_p)


def _edge_agg_body(t_hbm, src_hbm, msk_hbm, dst_hbm, out_hbm,
                   src_v, msk_v, dst_v, idx_v, rows_v, acc,
                   sem_st, sem_g, sem_sc):
    c = lax.axis_index("c")
    s = lax.axis_index("s")
    wbase = (c * NSUB + s) * ROWS_PER_W
    zeros16 = jnp.zeros((16,), jnp.float32)

    @pl.loop(0, ZROWS)
    def _(i):
        for q in range(DH // 16):
            rows_v[0, i, pl.ds(q * 16, 16)] = zeros16

    pltpu.sync_copy(rows_v.at[0], acc.at[pl.ds(s * ZROWS, ZROWS)])
    plsc.subcore_barrier()

    def stage(tt, p):
        b = tt * SUP + wbase
        return [pltpu.async_copy(src_hbm.at[pl.ds(b, SUP)], src_v.at[p], sem_st),
                pltpu.async_copy(msk_hbm.at[pl.ds(b, SUP)], msk_v.at[p], sem_st),
                pltpu.async_copy(dst_hbm.at[pl.ds(b, SUP)], dst_v.at[p], sem_st)]

    def idx_compute(p):
        for j in range(SUP):
            for q in range(CHUNK // 16):
                sl = pl.ds(q * 16, 16)
                idx_v[p, j, sl] = src_v[p, j, sl] + (1 - msk_v[p, j, sl]) * N_PAD

    def gathers(p):
        return [pltpu.async_copy(t_hbm.at[idx_v.at[p, j]],
                                 rows_v.at[p, pl.ds(j * CHUNK, CHUNK)], sem_g)
                for j in range(SUP)]

    def scatter(p, j):
        return pltpu.async_copy(rows_v.at[p, pl.ds(j * CHUNK, CHUNK)],
                                acc.at[dst_v.at[p, j]], sem_sc, add=True)

    @pl.loop(0, TSUP, step=2)
    def _(t):
        st0 = stage(t, 0)
        st1 = stage(t + 1, 1)
        for h in st0:
            h.wait()
        idx_compute(0)
        g0 = gathers(0)
        for h in st1:
            h.wait()
        idx_compute(1)
        g1 = gathers(1)
        scs = []
        for j in range(SUP):
            g0[j].wait()
            scs.append(scatter(0, j))
        for j in range(SUP):
            g1[j].wait()
            scs.append(scatter(1, j))
        for h in scs:
            h.wait()

    plsc.subcore_barrier()
    pltpu.sync_copy(acc.at[pl.ds(s * ZROWS, ZROWS)],
                    out_hbm.at[c, pl.ds(s * ZROWS, ZROWS)])


def _edge_agg(tbl, src_p, msk_p, dst_p):
    return pl.kernel(
        _edge_agg_body,
        out_type=jax.ShapeDtypeStruct((NCORE, N_PAD, DH), jnp.float32),
        mesh=_sc_mesh(),
        scratch_types=[
            pltpu.VMEM((2, SUP, CHUNK), jnp.int32),
            pltpu.VMEM((2, SUP, CHUNK), jnp.int32),
            pltpu.VMEM((2, SUP, CHUNK), jnp.int32),
            pltpu.VMEM((2, SUP, CHUNK), jnp.int32),
            pltpu.VMEM((2, SUP * CHUNK, DH), jnp.float32),
            pltpu.VMEM_SHARED((N_PAD, DH), jnp.float32),
            pltpu.SemaphoreType.DMA,
            pltpu.SemaphoreType.DMA,
            pltpu.SemaphoreType.DMA,
        ],
        compiler_params=pltpu.CompilerParams(use_tc_tiling_on_sc=False),
    )(tbl, src_p, msk_p, dst_p)


def _pack_outputs(y_e, y_o, bn, bo, t_ref, r_ref):
    t_ref[0] = jnp.concatenate([y_e[:, :DH] + bn, y_o[:, :DH] + bn], axis=1)
    t_ref[1] = jnp.concatenate([y_e[:, DH:2 * DH] + bo,
                                y_o[:, DH:2 * DH] + bo], axis=1)
    r_ref[...] = jnp.concatenate([y_e[:, 2 * DH:], y_o[:, 2 * DH:]], axis=1)


def _dense0_body(xe_ref, xo_ref, w_ref, bn_ref, bo_ref, t_ref, r_ref):
    y_e = jnp.dot(xe_ref[...], w_ref[...], preferred_element_type=jnp.float32)
    y_o = jnp.dot(xo_ref[...], w_ref[...], preferred_element_type=jnp.float32)
    _pack_outputs(y_e, y_o, bn_ref[...], bo_ref[...], t_ref, r_ref)


def _dense0(x_e, x_o, wc, bn, bo):
    hb = BLKD // 2
    return pl.pallas_call(
        _dense0_body,
        grid=(NBLKD,),
        in_specs=[
            pl.BlockSpec((hb, D_IN), lambda i: (i, 0)),
            pl.BlockSpec((hb, D_IN), lambda i: (i, 0)),
            pl.BlockSpec((D_IN, 3 * DH), lambda i: (0, 0)),
            pl.BlockSpec((1, DH), lambda i: (0, 0)),
            pl.BlockSpec((1, DH), lambda i: (0, 0)),
        ],
        out_specs=[
            pl.BlockSpec((2, hb, 2 * DH), lambda i: (0, i, 0)),
            pl.BlockSpec((hb, 2 * DH), lambda i: (i, 0)),
        ],
        out_shape=[
            jax.ShapeDtypeStruct((2, N_PAD // 2, 2 * DH), jnp.float32),
            jax.ShapeDtypeStruct((N_PAD // 2, 2 * DH), jnp.float32),
        ],
    )(x_e, x_o, wc, bn, bo)


def _post_agg(psum, inv, r, g, be):
    aggv = psum * inv + r
    mu = jnp.mean(aggv, axis=-1, keepdims=True)
    var = jnp.mean((aggv - mu) ** 2, axis=-1, keepdims=True)
    h = (aggv - mu) / jnp.sqrt(var + 1e-5) * g + be
    return jnp.maximum(h, 0.0)


def _halves(p_ref, ce_ref, co_ref, r_ref, g_ref, be_ref):
    p0, p1 = p_ref[0], p_ref[1]
    inv_e = 1.0 / jnp.maximum(ce_ref[0] + ce_ref[1], 1.0)
    inv_o = 1.0 / jnp.maximum(co_ref[0] + co_ref[1], 1.0)
    g, be = g_ref[...], be_ref[...]
    h_e = _post_agg(p0[:, :DH] + p1[:, :DH], inv_e, r_ref[:, :DH], g, be)
    h_o = _post_agg(p0[:, DH:] + p1[:, DH:], inv_o, r_ref[:, DH:], g, be)
    return h_e, h_o


def _combine_body(p_ref, ce_ref, co_ref, r_ref, g_ref, be_ref, w_ref,
                  bn_ref, bo_ref, t_ref, rn_ref):
    h_e, h_o = _halves(p_ref, ce_ref, co_ref, r_ref, g_ref, be_ref)
    y_e = jnp.dot(h_e, w_ref[...], preferred_element_type=jnp.float32)
    y_o = jnp.dot(h_o, w_ref[...], preferred_element_type=jnp.float32)
    _pack_outputs(y_e, y_o, bn_ref[...], bo_ref[...], t_ref, rn_ref)


def _combine(p2, ce, co, r2, g, be, wc, bn, bo):
    hb = BLKD // 2
    one = lambda i: (0, 0)
    return pl.pallas_call(
        _combine_body,
        grid=(NBLKD,),
        in_specs=[
            pl.BlockSpec((2, hb, 2 * DH), lambda i: (0, i, 0)),
            pl.BlockSpec((2, hb, 1), lambda i: (0, i, 0)),
            pl.BlockSpec((2, hb, 1), lambda i: (0, i, 0)),
            pl.BlockSpec((hb, 2 * DH), lambda i: (i, 0)),
            pl.BlockSpec((1, DH), one),
            pl.BlockSpec((1, DH), one),
            pl.BlockSpec((DH, 3 * DH), one),
            pl.BlockSpec((1, DH), one),
            pl.BlockSpec((1, DH), one),
        ],
        out_specs=[
            pl.BlockSpec((2, hb, 2 * DH), lambda i: (0, i, 0)),
            pl.BlockSpec((hb, 2 * DH), lambda i: (i, 0)),
        ],
        out_shape=[
            jax.ShapeDtypeStruct((2, N_PAD // 2, 2 * DH), jnp.float32),
            jax.ShapeDtypeStruct((N_PAD // 2, 2 * DH), jnp.float32),
        ],
    )(p2, ce, co, r2, g, be, wc, bn, bo)


def _dotT(a, b):
    return lax.dot_general(a, b, (((0,), (0,)), ((), ())),
                           preferred_element_type=jnp.float32)


def _final_body(p_ref, ce_ref, co_ref, r_ref, g_ref, be_ref, se_ref, so_ref,
                be2_ref, bo2_ref,
                wlin_ref, gbn_ref, bebn_ref, wo0_ref, wo1_ref, wo2_ref,
                boc_ref, bias_ref,
                outv_ref, xct_ref, loss_ref,
                out_acc, cs_acc):
    i = pl.program_id(0)
    hf = BLKF // 2
    h_e, h_o = _halves(p_ref, ce_ref, co_ref, r_ref, g_ref, be_ref)
    iota = lax.broadcasted_iota(jnp.int32, (hf, NGP), 1)
    bm_e = (be2_ref[...] == iota).astype(jnp.float32)
    bm_o = (bo2_ref[...] == iota).astype(jnp.float32)
    s_e = se_ref[...]
    s_o = so_ref[...]

    @pl.when(i == 0)
    def _():
        out_acc[...] = jnp.zeros_like(out_acc)
        cs_acc[...] = jnp.zeros_like(cs_acc)

    cs_acc[...] += _dotT(bm_e, s_e) + _dotT(bm_o, s_o)
    op_e = jnp.concatenate([s_e[:, c:c + 1] * h_e for c in range(NC)], axis=1)
    op_o = jnp.concatenate([s_o[:, c:c + 1] * h_o for c in range(NC)], axis=1)
    out_acc[...] += _dotT(bm_e, op_e) + _dotT(bm_o, op_o)

    @pl.when(i == NBLKF - 1)
    def _():
        cs = cs_acc[...]
        wl = wlin_ref[...]
        acc_outv = jnp.zeros((NGP, 1), jnp.float32) + bias_ref[0, 0]
        l1num = jnp.zeros((NGP, 1), jnp.float32)
        for c in range(NC):
            o = out_acc[:, c * DH:(c + 1) * DH]
            mu = jnp.mean(o, axis=-1, keepdims=True)
            var = jnp.mean((o - mu) ** 2, axis=-1, keepdims=True)
            y = (o - mu) / jnp.sqrt(var + 1e-5) * gbn_ref[...] + bebn_ref[...]
            xc = jnp.sum(y * wl, axis=-1, keepdims=True)
            xc = xc * (cs[:, c:c + 1] > 0).astype(jnp.float32)
            xct_ref[:, c:c + 1] = xc
            acc_outv += xc
            l1num += jnp.abs(xc)
        outv_ref[...] = acc_outv
        denom = jnp.sum((cs > 0).astype(jnp.float32) + 1e-7,
                        axis=-1, keepdims=True)
        rowm = (lax.broadcasted_iota(jnp.int32, (NGP, 1), 0) < NG)
        l1 = jnp.sum(rowm.astype(jnp.float32) * l1num / denom) / NG
        reg = (jnp.sum(jnp.abs(wo0_ref[...])) + jnp.sum(jnp.abs(wo1_ref[...]))
               + jnp.sum(jnp.abs(wo2_ref[...])) + jnp.sum(jnp.abs(boc_ref[...])))
        loss_ref[...] = jnp.reshape(0.01 * (reg + l1), (1, 1))


def _final(p2, ce, co, r2, g, be, s_e, s_o, b_e, b_o, wlin, gbn, bebn,
           wo0, wo1, wo2, boc, bias11):
    hf = BLKF // 2
    zero = lambda i: (0, 0)
    return pl.pallas_call(
        _final_body,
        grid=(NBLKF,),
        in_specs=[
            pl.BlockSpec((2, hf, 2 * DH), lambda i: (0, i, 0)),
            pl.BlockSpec((2, hf, 1), lambda i: (0, i, 0)),
            pl.BlockSpec((2, hf, 1), lambda i: (0, i, 0)),
            pl.BlockSpec((hf, 2 * DH), lambda i: (i, 0)),
            pl.BlockSpec((1, DH), zero),
            pl.BlockSpec((1, DH), zero),
            pl.BlockSpec((hf, NC), lambda i: (i, 0)),
            pl.BlockSpec((hf, NC), lambda i: (i, 0)),
            pl.BlockSpec((hf, 1), lambda i: (i, 0)),
            pl.BlockSpec((hf, 1), lambda i: (i, 0)),
            pl.BlockSpec((1, DH), zero),
            pl.BlockSpec((1, DH), zero),
            pl.BlockSpec((1, DH), zero),
            pl.BlockSpec((DH, D_IN), zero),
            pl.BlockSpec((DH, DH), zero),
            pl.BlockSpec((DH, DH), zero),
            pl.BlockSpec((1, 3 * DH), zero),
            pl.BlockSpec((1, 1), zero),
        ],
        out_specs=[
            pl.BlockSpec((NGP, 1), zero),
            pl.BlockSpec((NGP, NC), zero),
            pl.BlockSpec((1, 1), zero),
        ],
        out_shape=[
            jax.ShapeDtypeStruct((NGP, 1), jnp.float32),
            jax.ShapeDtypeStruct((NGP, NC), jnp.float32),
            jax.ShapeDtypeStruct((1, 1), jnp.float32),
        ],
        scratch_shapes=[
            pltpu.VMEM((NGP, NC * DH), jnp.float32),
            pltpu.VMEM((NGP, NC), jnp.float32),
        ],
        compiler_params=pltpu.CompilerParams(
            fuse_transposed_lhs_in_matmul=True),
    )(p2, ce, co, r2, g, be, s_e, s_o, b_e, b_o, wlin, gbn, bebn, wo0,
      wo1, wo2, boc, bias11)


def kernel(x, edge_index, s, batch, mask, W_n0, b_n0, W_o0, b_o0, W_r0, g0,
           be0, W_n1, b_n1, W_o1, b_o1, W_r1, g1, be1, W_n2, b_n2, W_o2,
           b_o2, W_r2, g2, be2, W_lin, bias, g_bn, be_bn):
    f32 = jnp.float32
    src = edge_index[0].astype(jnp.int32)
    dst = edge_index[1].astype(jnp.int32)
    mi = mask.astype(jnp.int32)

    pe = E_PAD - E
    lanes = jnp.arange(pe, dtype=jnp.int32) % CHUNK
    src_p = jnp.concatenate([src, lanes]).reshape(EROWS, CHUNK)
    dst_p = jnp.concatenate([dst, N + lanes]).reshape(EROWS, CHUNK)
    msk_p = jnp.concatenate([mi, jnp.ones((pe,), jnp.int32)]).reshape(
        EROWS, CHUNK)

    x_p = jnp.pad(x.astype(f32), ((0, N_PAD - N), (0, 0)))
    s_p = jnp.pad(s.astype(f32), ((0, N_PAD - N), (0, 0)))
    b_p = jnp.pad(batch.astype(jnp.int32), (0, N_PAD - N),
                  constant_values=NG)
    x_e, x_o = x_p[0::2], x_p[1::2]
    s_e, s_o = s_p[0::2], s_p[1::2]
    b_e = b_p[0::2].reshape(N_PAD // 2, 1)
    b_o = b_p[1::2].reshape(N_PAD // 2, 1)

    cntp = _degree(dst_p)
    src_p, msk_p, dst_p, cntp = lax.optimization_barrier(
        (src_p, msk_p, dst_p, cntp))
    cpair = cntp.reshape(2, N_PAD // 2, 2)
    ce = cpair[:, :, 0:1]
    co = cpair[:, :, 1:2]

    wc0 = jnp.concatenate([W_n0.T, W_o0.T, W_r0.T], axis=1)
    wc1 = jnp.concatenate([W_n1.T, W_o1.T, W_r1.T], axis=1)
    wc2 = jnp.concatenate([W_n2.T, W_o2.T, W_r2.T], axis=1)
    r1 = lambda v: v.reshape(1, -1)
    half = lambda p: p.reshape(2, N_PAD // 2, 2 * DH)

    t0, r0 = _dense0(x_e, x_o, wc0, r1(b_n0), r1(b_o0))
    p0 = _edge_agg(t0.reshape(2 * N_PAD, DH), src_p, msk_p, dst_p)
    t1, rr1 = _combine(half(p0), ce, co, r0, r1(g0), r1(be0), wc1,
                       r1(b_n1), r1(b_o1))
    p1 = _edge_agg(t1.reshape(2 * N_PAD, DH), src_p, msk_p, dst_p)
    t2, rr2 = _combine(half(p1), ce, co, rr1, r1(g1), r1(be1), wc2,
                       r1(b_n2), r1(b_o2))
    p2 = _edge_agg(t2.reshape(2 * N_PAD, DH), src_p, msk_p, dst_p)

    boc = jnp.concatenate([b_o0, b_o1, b_o2]).reshape(1, 3 * DH)
    outv_p, xct_p, loss = _final(half(p2), ce, co, rr2, r1(g2), r1(be2),
                                 s_e, s_o, b_e, b_o,
                                 W_lin.reshape(1, DH), r1(g_bn), r1(be_bn),
                                 W_o0, W_o1, W_o2, boc, bias.reshape(1, 1))

    outv = outv_p[:NG] + 0.0
    losses = loss.reshape(())
    xct = xct_p[:NG].reshape(NG, NC, 1)
    return outv, losses, s, xct
